# baseline (speedup 1.0000x reference)
import functools

import jax
import jax.numpy as jnp
from jax import lax
from jax.experimental import pallas as pl
from jax.experimental.pallas import tpu as pltpu
from jax.experimental.pallas import tpu_sc as plsc

N = 10000
E = 320000
D = 128
DT = 144
LANES = 16
NC = 2
NS = 16
NW = NC * NS
CHUNK = 100
NCHUNK = E // CHUNK
CPW = NCHUNK // NW
IBLK = 20
NBLK = CPW // IBLK
N_PAD = 10240
RPW = N_PAD // NS
ZROWS = 32


def _sc_body(xa_hbm, src_hbm, dst_hbm, t0_hbm, t1_hbm,
             idx_s, idx_d, rows0, rows1, zbuf, tbl_sh,
             sem_g0, sem_g1, sem_s0, sem_s1):
    c = lax.axis_index("c")
    s = lax.axis_index("s")
    wid = c * NS + s
    rows = (rows0, rows1)
    sem_g = (sem_g0, sem_g1)
    sem_s = (sem_s0, sem_s1)

    def zbuf_body(i, _):
        zbuf[i // (DT // LANES), pl.ds((i % (DT // LANES)) * LANES, LANES)] = (
            jnp.zeros((LANES,), jnp.float32))
        return 0
    lax.fori_loop(0, ZROWS * (DT // LANES), zbuf_body, 0)

    def z_s(k, _):
        pltpu.sync_copy(zbuf, tbl_sh.at[pl.ds(s * RPW + k * ZROWS, ZROWS)])
        return 0
    lax.fori_loop(0, RPW // ZROWS, z_s, 0)
    plsc.subcore_barrier()

    base = wid * CPW

    def block_body(b, _):
        pltpu.sync_copy(src_hbm.at[pl.ds(base + b * IBLK, IBLK)], idx_s)
        pltpu.sync_copy(dst_hbm.at[pl.ds(base + b * IBLK, IBLK)], idx_d)
        g = [None, None]
        sc = [None, None]
        g[0] = pltpu.async_copy(xa_hbm.at[idx_s.at[0]], rows0, sem_g0)
        for j in range(IBLK):
            p = j & 1
            q = 1 - p
            g[p].wait()
            if sc[q] is not None:
                sc[q].wait()
            if j + 1 < IBLK:
                g[q] = pltpu.async_copy(
                    xa_hbm.at[idx_s.at[j + 1]], rows[q], sem_g[q])
            sc[p] = pltpu.async_copy(
                rows[p], tbl_sh.at[idx_d.at[j]], sem_s[p], add=True)
        sc[(IBLK - 1) & 1].wait()
        return 0
    lax.fori_loop(0, NBLK, block_body, 0)
    plsc.subcore_barrier()

    @pl.when(c == 0)
    def _():
        pltpu.sync_copy(tbl_sh.at[pl.ds(s * RPW, RPW)],
                        t0_hbm.at[pl.ds(s * RPW, RPW)])

    @pl.when(c == 1)
    def _():
        pltpu.sync_copy(tbl_sh.at[pl.ds(s * RPW, RPW)],
                        t1_hbm.at[pl.ds(s * RPW, RPW)])


@functools.lru_cache(maxsize=1)
def _make_sc_segsum():
  return functools.partial(
    pl.kernel,
    out_type=(
        jax.ShapeDtypeStruct((N_PAD, DT), jnp.float32),
        jax.ShapeDtypeStruct((N_PAD, DT), jnp.float32),
    ),
    mesh=plsc.VectorSubcoreMesh(core_axis_name="c", subcore_axis_name="s",
                                num_cores=NC, num_subcores=NS),
    scratch_types=[
        pltpu.VMEM((IBLK, CHUNK), jnp.int32),
        pltpu.VMEM((IBLK, CHUNK), jnp.int32),
        pltpu.VMEM((CHUNK, DT), jnp.float32),
        pltpu.VMEM((CHUNK, DT), jnp.float32),
        pltpu.VMEM((ZROWS, DT), jnp.float32),
        pltpu.VMEM_SHARED((N_PAD, DT), jnp.float32),
        pltpu.SemaphoreType.DMA,
        pltpu.SemaphoreType.DMA,
        pltpu.SemaphoreType.DMA,
        pltpu.SemaphoreType.DMA,
    ],
    compiler_params=pltpu.CompilerParams(use_tc_tiling_on_sc=False),
  )(_sc_body)


def _tc_body(x_ref, t0_ref, t1_ref, we_ref, be_ref, wn_ref, bn_ref, out_ref):
    t0 = t0_ref[...]
    t1 = t1_ref[...]
    deg = t0[:, D:D + 1] + t1[:, D:D + 1]
    inv = 1.0 / jnp.maximum(deg, 1.0)
    mean_s = (t0[:, :D] + t1[:, :D]) * inv
    x = x_ref[...]
    w1 = we_ref[:, :D]
    w2 = we_ref[:, D:]
    t = (jnp.dot(mean_s, w1.T, preferred_element_type=jnp.float32)
         + jnp.dot(x, w2.T, preferred_element_type=jnp.float32)
         + be_ref[...])
    h = jnp.dot(t, wn_ref[...].T, preferred_element_type=jnp.float32) + bn_ref[...]
    out_ref[...] = jnp.where(deg > 0.0, h, x)


def _tc_dense(x, t0, t1, w_edge, b_edge, w_node, b_node):
    blk = 1000
    grid = (N // blk,)
    row_spec = pl.BlockSpec((blk, D), lambda i: (i, 0))
    tbl_spec = pl.BlockSpec((blk, DT), lambda i: (i, 0))
    full = lambda a, b: pl.BlockSpec((a, b), lambda i: (0, 0))
    return pl.pallas_call(
        _tc_body,
        grid=grid,
        in_specs=[
            row_spec, tbl_spec, tbl_spec,
            full(D, 2 * D), full(1, D), full(D, D), full(1, D),
        ],
        out_specs=row_spec,
        out_shape=jax.ShapeDtypeStruct((N, D), jnp.float32),
    )(x, t0, t1, w_edge, b_edge, w_node, b_node)


def kernel(node_inputs, edge_index, W_edge, b_edge, W_node, b_node):
    xa = jnp.concatenate(
        [node_inputs, jnp.ones((N, LANES), jnp.float32)], axis=1)
    src2 = edge_index[0].reshape(NCHUNK, CHUNK)
    dst2 = edge_index[1].reshape(NCHUNK, CHUNK)
    t0, t1 = _make_sc_segsum()(xa, src2, dst2)
    return _tc_dense(node_inputs, t0, t1, W_edge,
                     b_edge.reshape(1, D), W_node, b_node.reshape(1, D))


# speedup vs baseline: 1.2311x; 1.2311x over previous
import functools

import jax
import jax.numpy as jnp
from jax import lax
from jax.experimental import pallas as pl
from jax.experimental.pallas import tpu as pltpu
from jax.experimental.pallas import tpu_sc as plsc

N = 10000
E = 320000
D = 128
LANES = 16
NC = 2
NS = 16
NW = NC * NS
CHUNK = 100
NCHUNK = E // CHUNK
CPW = NCHUNK // NW
IBLK = 20
NBLK = CPW // IBLK
N_PAD = 10240
RPW = N_PAD // NS
ZROWS = 32


def _sc_body(x_hbm, src_hbm, dst_hbm, s0_hbm, s1_hbm, d0_hbm, d1_hbm,
             idx_s, idx_d, rows0, rows1, ones, zbuf, zdeg, s_sh, deg_sh,
             sem_g0, sem_g1, sem_s0, sem_s1, sem_o0, sem_o1):
    c = lax.axis_index("c")
    s = lax.axis_index("s")
    wid = c * NS + s
    rows = (rows0, rows1)
    sem_g = (sem_g0, sem_g1)
    sem_s = (sem_s0, sem_s1)
    sem_o = (sem_o0, sem_o1)

    def zbuf_body(i, _):
        zbuf[i // 8, pl.ds((i % 8) * LANES, LANES)] = jnp.zeros((LANES,), jnp.float32)
        return 0
    lax.fori_loop(0, ZROWS * 8, zbuf_body, 0)

    def zdeg_body(i, _):
        zdeg[i] = jnp.zeros((LANES,), jnp.float32)
        return 0
    lax.fori_loop(0, 64, zdeg_body, 0)

    def ones_body(i, _):
        ones[i] = jnp.ones((LANES,), jnp.float32)
        return 0
    lax.fori_loop(0, CHUNK, ones_body, 0)

    def z_s(k, _):
        pltpu.sync_copy(zbuf, s_sh.at[pl.ds(s * RPW + k * ZROWS, ZROWS)])
        return 0
    lax.fori_loop(0, RPW // ZROWS, z_s, 0)

    def z_d(k, _):
        pltpu.sync_copy(zdeg, deg_sh.at[pl.ds(s * RPW + k * 64, 64)])
        return 0
    lax.fori_loop(0, RPW // 64, z_d, 0)
    plsc.subcore_barrier()

    base = wid * CPW

    def block_body(b, _):
        pltpu.sync_copy(src_hbm.at[pl.ds(base + b * IBLK, IBLK)], idx_s)
        pltpu.sync_copy(dst_hbm.at[pl.ds(base + b * IBLK, IBLK)], idx_d)
        g = [None, None]
        sc = [None, None]
        oc = [None, None]
        g[0] = pltpu.async_copy(x_hbm.at[idx_s.at[0]], rows0, sem_g0)
        for j in range(IBLK):
            p = j & 1
            q = 1 - p
            g[p].wait()
            if sc[q] is not None:
                sc[q].wait()
                oc[q].wait()
            if j + 1 < IBLK:
                g[q] = pltpu.async_copy(
                    x_hbm.at[idx_s.at[j + 1]], rows[q], sem_g[q])
            sc[p] = pltpu.async_copy(
                rows[p], s_sh.at[idx_d.at[j]], sem_s[p], add=True)
            oc[p] = pltpu.async_copy(
                ones, deg_sh.at[idx_d.at[j]], sem_o[p], add=True)
        last = (IBLK - 1) & 1
        sc[last].wait()
        oc[last].wait()
        return 0
    lax.fori_loop(0, NBLK, block_body, 0)
    plsc.subcore_barrier()

    @pl.when(c == 0)
    def _():
        pltpu.sync_copy(s_sh.at[pl.ds(s * RPW, RPW)],
                        s0_hbm.at[pl.ds(s * RPW, RPW)])
        pltpu.sync_copy(deg_sh.at[pl.ds(s * RPW, RPW)],
                        d0_hbm.at[pl.ds(s * RPW, RPW)])

    @pl.when(c == 1)
    def _():
        pltpu.sync_copy(s_sh.at[pl.ds(s * RPW, RPW)],
                        s1_hbm.at[pl.ds(s * RPW, RPW)])
        pltpu.sync_copy(deg_sh.at[pl.ds(s * RPW, RPW)],
                        d1_hbm.at[pl.ds(s * RPW, RPW)])


@functools.lru_cache(maxsize=1)
def _make_sc_segsum():
  return functools.partial(
    pl.kernel,
    out_type=(
        jax.ShapeDtypeStruct((N_PAD, D), jnp.float32),
        jax.ShapeDtypeStruct((N_PAD, D), jnp.float32),
        jax.ShapeDtypeStruct((N_PAD, LANES), jnp.float32),
        jax.ShapeDtypeStruct((N_PAD, LANES), jnp.float32),
    ),
    mesh=plsc.VectorSubcoreMesh(core_axis_name="c", subcore_axis_name="s",
                                num_cores=NC, num_subcores=NS),
    scratch_types=[
        pltpu.VMEM((IBLK, CHUNK), jnp.int32),
        pltpu.VMEM((IBLK, CHUNK), jnp.int32),
        pltpu.VMEM((CHUNK, D), jnp.float32),
        pltpu.VMEM((CHUNK, D), jnp.float32),
        pltpu.VMEM((CHUNK, LANES), jnp.float32),
        pltpu.VMEM((ZROWS, D), jnp.float32),
        pltpu.VMEM((64, LANES), jnp.float32),
        pltpu.VMEM_SHARED((N_PAD, D), jnp.float32),
        pltpu.VMEM_SHARED((N_PAD, LANES), jnp.float32),
        pltpu.SemaphoreType.DMA,
        pltpu.SemaphoreType.DMA,
        pltpu.SemaphoreType.DMA,
        pltpu.SemaphoreType.DMA,
        pltpu.SemaphoreType.DMA,
        pltpu.SemaphoreType.DMA,
    ],
    compiler_params=pltpu.CompilerParams(use_tc_tiling_on_sc=False),
  )(_sc_body)


def _tc_a_body(x_ref, we_ref, be_ref, wn_ref, bn_ref, z_ref, m1t_ref):
    w1 = we_ref[:, :D]
    w2 = we_ref[:, D:]
    wnt = wn_ref[...].T
    zx = jnp.dot(x_ref[...], w2.T, preferred_element_type=jnp.float32) + be_ref[...]
    z_ref[...] = jnp.dot(zx, wnt, preferred_element_type=jnp.float32) + bn_ref[...]
    m1t_ref[...] = jnp.dot(w1.T, wnt, preferred_element_type=jnp.float32)


def _tc_a(x, w_edge, b_edge, w_node, b_node):
    blk = 1000
    row_spec = pl.BlockSpec((blk, D), lambda i: (i, 0))
    full = lambda a, b: pl.BlockSpec((a, b), lambda i: (0, 0))
    return pl.pallas_call(
        _tc_a_body,
        grid=(N // blk,),
        in_specs=[row_spec, full(D, 2 * D), full(1, D), full(D, D), full(1, D)],
        out_specs=[row_spec, full(D, D)],
        out_shape=[
            jax.ShapeDtypeStruct((N, D), jnp.float32),
            jax.ShapeDtypeStruct((D, D), jnp.float32),
        ],
    )(x, w_edge, b_edge, w_node, b_node)


def _tc_b_body(x_ref, s0_ref, s1_ref, d0_ref, d1_ref, z_ref, m1t_ref, out_ref):
    deg = d0_ref[:, 0:1] + d1_ref[:, 0:1]
    inv = 1.0 / jnp.maximum(deg, 1.0)
    mean_s = (s0_ref[...] + s1_ref[...]) * inv
    h = jnp.dot(mean_s, m1t_ref[...], preferred_element_type=jnp.float32) + z_ref[...]
    out_ref[...] = jnp.where(deg > 0.0, h, x_ref[...])


def _tc_b(x, s0, s1, d0, d1, z, m1t):
    blk = 1000
    row_spec = pl.BlockSpec((blk, D), lambda i: (i, 0))
    deg_spec = pl.BlockSpec((blk, LANES), lambda i: (i, 0))
    full = lambda a, b: pl.BlockSpec((a, b), lambda i: (0, 0))
    return pl.pallas_call(
        _tc_b_body,
        grid=(N // blk,),
        in_specs=[row_spec, row_spec, row_spec, deg_spec, deg_spec,
                  row_spec, full(D, D)],
        out_specs=row_spec,
        out_shape=jax.ShapeDtypeStruct((N, D), jnp.float32),
    )(x, s0, s1, d0, d1, z, m1t)


def kernel(node_inputs, edge_index, W_edge, b_edge, W_node, b_node):
    src2 = edge_index[0].reshape(NCHUNK, CHUNK)
    dst2 = edge_index[1].reshape(NCHUNK, CHUNK)
    s0, s1, d0, d1 = _make_sc_segsum()(node_inputs, src2, dst2)
    z, m1t = _tc_a(node_inputs, W_edge, b_edge.reshape(1, D),
                   W_node, b_node.reshape(1, D))
    return _tc_b(node_inputs, s0, s1, d0, d1, z, m1t)
